# baseline (speedup 1.0000x reference)
import functools

import jax
import jax.numpy as jnp
from jax import lax
from jax.experimental import pallas as pl
from jax.experimental.pallas import tpu as pltpu
from jax.experimental.pallas import tpu_sc as plsc

N = 10000
E = 320000
F_IN = 128
HID = 16
HEADS = 8
NCLS = 16

NC, NS = 2, 16
NW = NC * NS
EDGES_PER_W = 10240
EPAD = NW * EDGES_PER_W
NPAD = 10112
ROWS_PER_TILE = NPAD // NS

W1COLS = HEADS * HID
XPA1_W = W1COLS + 16
ACC1_W = XPA1_W
XPA2_W = NCLS + 16
ACC2_W = XPA2_W

_R = 2000
_GRID = N // _R


def _lrelu(v):
    return jnp.where(v >= 0, v, 0.2 * v)


def _phase_a_body(x_ref, w1_ref, a1s_ref, a1d_ref, xpa_ref, adst_ref, half_ref):
    xp = jnp.dot(x_ref[...], w1_ref[...], preferred_element_type=jnp.float32)
    asrc = jnp.dot(xp, a1s_ref[...], preferred_element_type=jnp.float32)
    adst = jnp.dot(xp, a1d_ref[...], preferred_element_type=jnp.float32)
    colh = lax.broadcasted_iota(jnp.int32, (HEADS, W1COLS), 1) // HID
    rowh = lax.broadcasted_iota(jnp.int32, (HEADS, W1COLS), 0)
    rep = (colh == rowh).astype(jnp.float32)
    wself = jnp.exp(_lrelu(asrc + adst))
    wrep = jnp.dot(wself, rep, preferred_element_type=jnp.float32)
    z8 = jnp.zeros((xp.shape[0], 8), jnp.float32)
    xpa_ref[...] = jnp.concatenate([xp, asrc, z8], axis=1)
    adst_ref[...] = jnp.concatenate([adst, z8], axis=1)
    half_ref[...] = jnp.concatenate(
        [xp * wrep * 0.5, wself * 0.5, z8], axis=1)


def _phase_a(x, W1, A1s, A1d):
    return pl.pallas_call(
        _phase_a_body,
        grid=(_GRID,),
        in_specs=[
            pl.BlockSpec((_R, F_IN), lambda i: (i, 0)),
            pl.BlockSpec((F_IN, W1COLS), lambda i: (0, 0)),
            pl.BlockSpec((W1COLS, HEADS), lambda i: (0, 0)),
            pl.BlockSpec((W1COLS, HEADS), lambda i: (0, 0)),
        ],
        out_specs=[
            pl.BlockSpec((_R, XPA1_W), lambda i: (i, 0)),
            pl.BlockSpec((_R, 16), lambda i: (i, 0)),
            pl.BlockSpec((_R, XPA1_W), lambda i: (i, 0)),
        ],
        out_shape=[
            jax.ShapeDtypeStruct((N, XPA1_W), jnp.float32),
            jax.ShapeDtypeStruct((N, 16), jnp.float32),
            jax.ShapeDtypeStruct((N, XPA1_W), jnp.float32),
        ],
    )(x, W1, A1s, A1d)


def _bcast_lane(v, j):
    idx = jnp.full((16,), j, dtype=jnp.int32)
    dn = lax.GatherDimensionNumbers(
        offset_dims=(), collapsed_slice_dims=(0,), start_index_map=(0,))
    return lax.gather(v, idx[:, None], dn, (1,),
                      mode=lax.GatherScatterMode.PROMISE_IN_BOUNDS)


def _make_sc_edge(width, nheads, k, g0_frac=0.5):
    mesh = plsc.VectorSubcoreMesh(core_axis_name="c", subcore_axis_name="s")
    tot = 2 * (EDGES_PER_W // k)
    g0 = max(4, int(round(g0_frac * tot / 4)) * 4)
    g1 = tot - g0
    assert g1 >= 4 and g1 % 4 == 0

    @functools.partial(
        pl.kernel,
        out_type=jax.ShapeDtypeStruct((NC, NPAD, width), jnp.float32),
        mesh=mesh,
        scratch_types=[
            pltpu.VMEM_SHARED((NPAD, width), jnp.float32),
            pltpu.VMEM((4, k), jnp.int32),
            pltpu.VMEM((4, k), jnp.int32),
            pltpu.VMEM((2, k, width), jnp.float32),
            pltpu.VMEM((2, k, 16), jnp.float32),
            pltpu.VMEM((2, k, width), jnp.float32),
            [pltpu.SemaphoreType.DMA] * 2,
            [pltpu.SemaphoreType.DMA] * 2,
            [pltpu.SemaphoreType.DMA] * 2,
            [pltpu.SemaphoreType.DMA] * 4,
        ],
        compiler_params=pltpu.CompilerParams(use_tc_tiling_on_sc=False),
    )
    def sc_edge(src_hbm, dst_hbm, xpa_hbm, adst_hbm, half_hbm, out_hbm,
                acc, sidx, didx, xbuf, abuf, obuf, gx, ga, sc, isem):
        c = lax.axis_index("c")
        s = lax.axis_index("s")
        r0 = s * ROWS_PER_TILE
        row0 = jnp.where(c == 0, s * g0, NS * g0 + s * g1)
        niter = jnp.where(c == 0, g0 // 4, g1 // 4)

        def load_idx(ci, islot):
            pltpu.async_copy(src_hbm.at[row0 + ci], sidx.at[islot],
                             isem[islot])
            pltpu.async_copy(dst_hbm.at[row0 + ci], didx.at[islot],
                             isem[islot])

        def wait_idx(ci, islot):
            pltpu.make_async_copy(src_hbm.at[row0 + ci], sidx.at[islot],
                                  isem[islot]).wait()
            pltpu.make_async_copy(dst_hbm.at[row0 + ci], didx.at[islot],
                                  isem[islot]).wait()

        def issue_gather(islot, bslot):
            pltpu.async_copy(xpa_hbm.at[sidx.at[islot]], xbuf.at[bslot],
                             gx[bslot])
            pltpu.async_copy(adst_hbm.at[didx.at[islot]], abuf.at[bslot],
                             ga[bslot])

        def wait_gather(islot, bslot):
            pltpu.make_async_copy(xpa_hbm.at[sidx.at[islot]], xbuf.at[bslot],
                                  gx[bslot]).wait()
            pltpu.make_async_copy(adst_hbm.at[didx.at[islot]],
                                  abuf.at[bslot], ga[bslot]).wait()

        def issue_scatter(islot, bslot):
            pltpu.async_copy(obuf.at[bslot], acc.at[didx.at[islot]],
                             sc[bslot], add=True)

        def wait_scatter(islot, bslot):
            pltpu.make_async_copy(obuf.at[bslot], acc.at[didx.at[islot]],
                                  sc[bslot]).wait()

        def compute(bslot):
            @plsc.parallel_loop(0, k, 1, unroll=4)
            def edge_body(e):
                av = xbuf[bslot, e, pl.ds(width - 16, 16)] + abuf[bslot, e, :]
                wv = jnp.exp(_lrelu(av))
                obuf[bslot, e, pl.ds(width - 16, 16)] = wv
                for j in range(nheads):
                    ws = _bcast_lane(wv, j)
                    obuf[bslot, e, pl.ds(j * 16, 16)] = (
                        ws * xbuf[bslot, e, pl.ds(j * 16, 16)])

        load_idx(0, 0)
        load_idx(1, 1)
        nstage, rem = divmod(ROWS_PER_TILE, k)
        for t in range(nstage):
            rr = r0 + t * k
            stg = obuf.at[t % 2]
            pltpu.sync_copy(half_hbm.at[pl.ds(rr, k), :], stg)
            pltpu.sync_copy(stg, acc.at[pl.ds(rr, k), :])
        if rem:
            rr = r0 + nstage * k
            stg = obuf.at[nstage % 2]
            pltpu.sync_copy(half_hbm.at[pl.ds(rr, rem), :],
                            stg.at[pl.ds(0, rem), :])
            pltpu.sync_copy(stg.at[pl.ds(0, rem), :],
                            acc.at[pl.ds(rr, rem), :])
        plsc.subcore_barrier()
        wait_idx(0, 0)
        issue_gather(0, 0)

        def quad_body(g, _):
            base = 4 * g
            for j in range(4):
                ci = base + j
                bslot = j % 2
                if j < 3:
                    wait_idx(ci + 1, (j + 1) % 4)
                    issue_gather((j + 1) % 4, 1 - bslot)
                else:
                    @pl.when(g + 1 < niter)
                    def _():
                        wait_idx(ci + 1, (j + 1) % 4)
                        issue_gather((j + 1) % 4, 1 - bslot)

                wait_gather(j, bslot)
                if j < 2:
                    @pl.when(g >= 1)
                    def _():
                        wait_scatter((j + 2) % 4, bslot)
                else:
                    wait_scatter((j + 2) % 4, bslot)
                if j < 2:
                    load_idx(ci + 2, (j + 2) % 4)
                else:
                    @pl.when(g + 1 < niter)
                    def _():
                        load_idx(ci + 2, (j + 2) % 4)

                compute(bslot)
                issue_scatter(j, bslot)
            return 0

        lax.fori_loop(0, niter, quad_body, 0)
        wait_scatter(2, 0)
        wait_scatter(3, 1)
        plsc.subcore_barrier()
        for t in range(nstage):
            rr = r0 + t * k
            stg = obuf.at[t % 2]
            pltpu.sync_copy(acc.at[pl.ds(rr, k), :], stg)
            pltpu.sync_copy(stg, out_hbm.at[c, pl.ds(rr, k), :])
        if rem:
            rr = r0 + nstage * k
            stg = obuf.at[nstage % 2]
            pltpu.sync_copy(acc.at[pl.ds(rr, rem), :],
                            stg.at[pl.ds(0, rem), :])
            pltpu.sync_copy(stg.at[pl.ds(0, rem), :],
                            out_hbm.at[c, pl.ds(rr, rem), :])

    return sc_edge


def _make_sc_edge_deep(width, nheads, k, ring, g0_frac=0.5):
    mesh = plsc.VectorSubcoreMesh(core_axis_name="c", subcore_axis_name="s")
    tot = 2 * (EDGES_PER_W // k)
    g0 = int(round(g0_frac * tot / ring)) * ring
    g1 = tot - g0
    assert g0 >= ring and g1 >= ring and g1 % ring == 0
    chunks = g0

    @functools.partial(
        pl.kernel,
        out_type=jax.ShapeDtypeStruct((NC, NPAD, width), jnp.float32),
        mesh=mesh,
        scratch_types=[
            pltpu.VMEM_SHARED((NPAD, width), jnp.float32),
            pltpu.VMEM((chunks, k), jnp.int32),
            pltpu.VMEM((chunks, k), jnp.int32),
            pltpu.VMEM((ring, k, width), jnp.float32),
            pltpu.VMEM((ring, k, 16), jnp.float32),
            pltpu.VMEM((2, k, width), jnp.float32),
            [pltpu.SemaphoreType.DMA] * ring,
            [pltpu.SemaphoreType.DMA] * ring,
            [pltpu.SemaphoreType.DMA] * 2,
        ],
        compiler_params=pltpu.CompilerParams(use_tc_tiling_on_sc=False),
    )
    def sc_edge(src_hbm, dst_hbm, xpa_hbm, adst_hbm, half_hbm, out_hbm,
                acc, sidx, didx, xbuf, abuf, obuf, gx, ga, sc):
        c = lax.axis_index("c")
        s = lax.axis_index("s")
        r0 = s * ROWS_PER_TILE
        row0 = jnp.where(c == 0, s * g0, NS * g0 + s * g1)
        my_chunks = jnp.where(c == 0, g0, g1)
        niter = my_chunks // ring

        def issue_gather(ci, slot):
            pltpu.async_copy(xpa_hbm.at[sidx.at[ci]], xbuf.at[slot],
                             gx[slot])
            pltpu.async_copy(adst_hbm.at[didx.at[ci]], abuf.at[slot],
                             ga[slot])

        def wait_gather(ci, slot):
            pltpu.make_async_copy(xpa_hbm.at[sidx.at[ci]], xbuf.at[slot],
                                  gx[slot]).wait()
            pltpu.make_async_copy(adst_hbm.at[didx.at[ci]],
                                  abuf.at[slot], ga[slot]).wait()

        def issue_scatter(ci, oslot):
            pltpu.async_copy(obuf.at[oslot], acc.at[didx.at[ci]],
                             sc[oslot], add=True)

        def wait_scatter(ci, oslot):
            pltpu.make_async_copy(obuf.at[oslot], acc.at[didx.at[ci]],
                                  sc[oslot]).wait()

        def compute(slot, oslot):
            @plsc.parallel_loop(0, k, 1, unroll=4)
            def edge_body(e):
                av = xbuf[slot, e, pl.ds(width - 16, 16)] + abuf[slot, e, :]
                wv = jnp.exp(_lrelu(av))
                obuf[oslot, e, pl.ds(width - 16, 16)] = wv
                for j in range(nheads):
                    ws = _bcast_lane(wv, j)
                    obuf[oslot, e, pl.ds(j * 16, 16)] = (
                        ws * xbuf[slot, e, pl.ds(j * 16, 16)])

        @pl.when(c == 0)
        def _():
            pltpu.sync_copy(src_hbm.at[pl.ds(row0, g0), :], sidx)
            pltpu.sync_copy(dst_hbm.at[pl.ds(row0, g0), :], didx)

        @pl.when(c == 1)
        def _():
            pltpu.sync_copy(src_hbm.at[pl.ds(row0, g1), :],
                            sidx.at[pl.ds(0, g1), :])
            pltpu.sync_copy(dst_hbm.at[pl.ds(row0, g1), :],
                            didx.at[pl.ds(0, g1), :])
        nstage, rem = divmod(ROWS_PER_TILE, 2 * k)
        for t in range(nstage):
            rr = r0 + t * 2 * k
            pltpu.sync_copy(half_hbm.at[pl.ds(rr, k), :], obuf.at[0])
            pltpu.sync_copy(half_hbm.at[pl.ds(rr + k, k), :], obuf.at[1])
            pltpu.sync_copy(obuf.at[0], acc.at[pl.ds(rr, k), :])
            pltpu.sync_copy(obuf.at[1], acc.at[pl.ds(rr + k, k), :])
        if rem:
            rr = r0 + nstage * 2 * k
            pltpu.sync_copy(half_hbm.at[pl.ds(rr, rem), :],
                            obuf.at[0].at[pl.ds(0, rem), :])
            pltpu.sync_copy(obuf.at[0].at[pl.ds(0, rem), :],
                            acc.at[pl.ds(rr, rem), :])
        plsc.subcore_barrier()

        for j in range(ring - 2):
            issue_gather(j, j)

        def ring_body(g, _):
            base = ring * g
            for j in range(ring):
                ci = base + j
                oslot = j % 2
                wait_gather(ci, j)
                if j < 2:
                    @pl.when(g >= 1)
                    def _():
                        wait_scatter(ci - 2, oslot)
                else:
                    wait_scatter(ci - 2, oslot)
                nci = ci + ring - 2

                @pl.when(nci < my_chunks)
                def _():
                    issue_gather(nci, (j + ring - 2) % ring)

                compute(j, oslot)
                issue_scatter(ci, oslot)
            return 0

        lax.fori_loop(0, niter, ring_body, 0)
        wait_scatter(0, 0)
        wait_scatter(1, 1)
        plsc.subcore_barrier()
        for t in range(nstage):
            rr = r0 + t * 2 * k
            pltpu.sync_copy(acc.at[pl.ds(rr, k), :], obuf.at[0])
            pltpu.sync_copy(acc.at[pl.ds(rr + k, k), :], obuf.at[1])
            pltpu.sync_copy(obuf.at[0], out_hbm.at[c, pl.ds(rr, k), :])
            pltpu.sync_copy(obuf.at[1], out_hbm.at[c, pl.ds(rr + k, k), :])
        if rem:
            rr = r0 + nstage * 2 * k
            pltpu.sync_copy(acc.at[pl.ds(rr, rem), :],
                            obuf.at[0].at[pl.ds(0, rem), :])
            pltpu.sync_copy(obuf.at[0].at[pl.ds(0, rem), :],
                            out_hbm.at[c, pl.ds(rr, rem), :])

    return sc_edge


_sc_edge1 = _make_sc_edge(ACC1_W, HEADS, 64, g0_frac=0.9875)
_sc_edge2 = _make_sc_edge_deep(ACC2_W, 1, 128, 8, g0_frac=0.95)


def _phase_c_body(acc_ref, w2_ref, a2s_ref, a2d_ref, b1_ref,
                  xpa_ref, adst_ref, half_ref):
    num = acc_ref[0, :, 0:W1COLS] + acc_ref[1, :, 0:W1COLS]
    den = acc_ref[0, :, W1COLS:W1COLS + 8] + acc_ref[1, :, W1COLS:W1COLS + 8]
    colh = lax.broadcasted_iota(jnp.int32, (HEADS, W1COLS), 1) // HID
    rowh = lax.broadcasted_iota(jnp.int32, (HEADS, W1COLS), 0)
    rep = (colh == rowh).astype(jnp.float32)
    denr = jnp.dot(den, rep, preferred_element_type=jnp.float32)
    h = num / (denr + 1e-16) + b1_ref[...]
    h = jnp.where(h > 0, h, jnp.exp(jnp.minimum(h, 0.0)) - 1.0)
    xp2 = jnp.dot(h, w2_ref[...], preferred_element_type=jnp.float32)
    asrc2 = jnp.dot(xp2, a2s_ref[...], preferred_element_type=jnp.float32)
    adst2 = jnp.dot(xp2, a2d_ref[...], preferred_element_type=jnp.float32)
    wself = jnp.exp(_lrelu(asrc2 + adst2))
    z15 = jnp.zeros((h.shape[0], 15), jnp.float32)
    xpa_ref[...] = jnp.concatenate([xp2, asrc2, z15], axis=1)
    adst_ref[...] = jnp.concatenate([adst2, z15], axis=1)
    half_ref[...] = jnp.concatenate([xp2 * wself * 0.5, wself * 0.5, z15],
                                    axis=1)


def _phase_c(acc1, W2, a2s, a2d, b1r):
    return pl.pallas_call(
        _phase_c_body,
        grid=(_GRID,),
        in_specs=[
            pl.BlockSpec((NC, _R, ACC1_W), lambda i: (0, i, 0)),
            pl.BlockSpec((W1COLS, NCLS), lambda i: (0, 0)),
            pl.BlockSpec((NCLS, 1), lambda i: (0, 0)),
            pl.BlockSpec((NCLS, 1), lambda i: (0, 0)),
            pl.BlockSpec((1, W1COLS), lambda i: (0, 0)),
        ],
        out_specs=[
            pl.BlockSpec((_R, XPA2_W), lambda i: (i, 0)),
            pl.BlockSpec((_R, 16), lambda i: (i, 0)),
            pl.BlockSpec((_R, XPA2_W), lambda i: (i, 0)),
        ],
        out_shape=[
            jax.ShapeDtypeStruct((N, XPA2_W), jnp.float32),
            jax.ShapeDtypeStruct((N, 16), jnp.float32),
            jax.ShapeDtypeStruct((N, XPA2_W), jnp.float32),
        ],
    )(acc1, W2, a2s, a2d, b1r)


def _phase_e_body(acc_ref, b2_ref, out_ref):
    num = acc_ref[0, :, 0:NCLS] + acc_ref[1, :, 0:NCLS]
    den = acc_ref[0, :, NCLS:NCLS + 1] + acc_ref[1, :, NCLS:NCLS + 1]
    o = num / (den + 1e-16) + b2_ref[...]
    m = jnp.max(o, axis=1, keepdims=True)
    sh = o - m
    lse = jnp.log(jnp.sum(jnp.exp(sh), axis=1, keepdims=True))
    out_ref[...] = sh - lse


def _phase_e(acc2, b2r):
    return pl.pallas_call(
        _phase_e_body,
        grid=(_GRID,),
        in_specs=[
            pl.BlockSpec((NC, _R, ACC2_W), lambda i: (0, i, 0)),
            pl.BlockSpec((1, NCLS), lambda i: (0, 0)),
        ],
        out_specs=pl.BlockSpec((_R, NCLS), lambda i: (i, 0)),
        out_shape=jax.ShapeDtypeStruct((N, NCLS), jnp.float32),
    )(acc2, b2r)


def kernel(x, edge_index, W1, att_src1, att_dst1, b1, W2, att_src2,
           att_dst2, b2):
    src = edge_index[0].astype(jnp.int32)
    dst = edge_index[1].astype(jnp.int32)
    npad_e = EPAD - E
    src_p = jnp.concatenate([src, jnp.zeros((npad_e,), jnp.int32)])
    dst_p = jnp.concatenate([dst, jnp.full((npad_e,), N, jnp.int32)])

    a1s_flat = att_src1.reshape(W1COLS)
    a1d_flat = att_dst1.reshape(W1COLS)
    heads_of = jnp.arange(W1COLS) // HID
    onehot = jax.nn.one_hot(heads_of, HEADS, dtype=jnp.float32)
    A1s = a1s_flat[:, None] * onehot
    A1d = a1d_flat[:, None] * onehot
    a2s = att_src2.reshape(NCLS, 1)
    a2d = att_dst2.reshape(NCLS, 1)
    b1r = b1.reshape(1, W1COLS)
    b2r = b2.reshape(1, NCLS)

    xpa1, adst1, half1 = _phase_a(x, W1, A1s, A1d)
    adst1p = jnp.concatenate(
        [adst1, jnp.zeros((NPAD - N, 16), jnp.float32)], axis=0)
    half1p = jnp.concatenate(
        [half1, jnp.zeros((NPAD - N, ACC1_W), jnp.float32)], axis=0)
    acc1 = _sc_edge1(src_p.reshape(-1, 64), dst_p.reshape(-1, 64),
                     xpa1, adst1p, half1p)

    xpa2, adst2, half2 = _phase_c(acc1, W2, a2s, a2d, b1r)
    adst2p = jnp.concatenate(
        [adst2, jnp.zeros((NPAD - N, 16), jnp.float32)], axis=0)
    half2p = jnp.concatenate(
        [half2, jnp.zeros((NPAD - N, ACC2_W), jnp.float32)], axis=0)
    acc2 = _sc_edge2(src_p.reshape(-1, 128), dst_p.reshape(-1, 128),
                     xpa2, adst2p, half2p)

    return _phase_e(acc2, b2r)


# speedup vs baseline: 1.0915x; 1.0915x over previous
import functools

import jax
import jax.numpy as jnp
from jax import lax
from jax.experimental import pallas as pl
from jax.experimental.pallas import tpu as pltpu
from jax.experimental.pallas import tpu_sc as plsc

N = 10000
E = 320000
F_IN = 128
HID = 16
HEADS = 8
NCLS = 16

NC, NS = 2, 16
NW = NC * NS
EDGES_PER_W = 10240
EPAD = NW * EDGES_PER_W
NPAD = 10112
ROWS_PER_TILE = NPAD // NS

W1COLS = HEADS * HID
XPA1_W = W1COLS + 16
ACC1_W = XPA1_W
XPA2_W = NCLS + 16
ACC2_W = XPA2_W

_R = 2000
_GRID = N // _R


def _lrelu(v):
    return jnp.where(v >= 0, v, 0.2 * v)


def _phase_a_body(x_ref, w1_ref, a1s_ref, a1d_ref, xpa_ref, adst_ref, half_ref):
    xp = jnp.dot(x_ref[...], w1_ref[...], preferred_element_type=jnp.float32)
    asrc = jnp.dot(xp, a1s_ref[...], preferred_element_type=jnp.float32)
    adst = jnp.dot(xp, a1d_ref[...], preferred_element_type=jnp.float32)
    colh = lax.broadcasted_iota(jnp.int32, (HEADS, W1COLS), 1) // HID
    rowh = lax.broadcasted_iota(jnp.int32, (HEADS, W1COLS), 0)
    rep = (colh == rowh).astype(jnp.float32)
    wself = jnp.exp(_lrelu(asrc + adst))
    wrep = jnp.dot(wself, rep, preferred_element_type=jnp.float32)
    z8 = jnp.zeros((xp.shape[0], 8), jnp.float32)
    xpa_ref[...] = jnp.concatenate([xp, asrc, z8], axis=1)
    adst_ref[...] = jnp.concatenate([adst, z8], axis=1)
    half_ref[...] = jnp.concatenate(
        [xp * wrep * 0.5, wself * 0.5, z8], axis=1)


def _phase_a(x, W1, A1s, A1d):
    return pl.pallas_call(
        _phase_a_body,
        grid=(_GRID,),
        in_specs=[
            pl.BlockSpec((_R, F_IN), lambda i: (i, 0)),
            pl.BlockSpec((F_IN, W1COLS), lambda i: (0, 0)),
            pl.BlockSpec((W1COLS, HEADS), lambda i: (0, 0)),
            pl.BlockSpec((W1COLS, HEADS), lambda i: (0, 0)),
        ],
        out_specs=[
            pl.BlockSpec((_R, XPA1_W), lambda i: (i, 0)),
            pl.BlockSpec((_R, 16), lambda i: (i, 0)),
            pl.BlockSpec((_R, XPA1_W), lambda i: (i, 0)),
        ],
        out_shape=[
            jax.ShapeDtypeStruct((N, XPA1_W), jnp.float32),
            jax.ShapeDtypeStruct((N, 16), jnp.float32),
            jax.ShapeDtypeStruct((N, XPA1_W), jnp.float32),
        ],
    )(x, W1, A1s, A1d)


def _bcast_lane(v, j):
    idx = jnp.full((16,), j, dtype=jnp.int32)
    dn = lax.GatherDimensionNumbers(
        offset_dims=(), collapsed_slice_dims=(0,), start_index_map=(0,))
    return lax.gather(v, idx[:, None], dn, (1,),
                      mode=lax.GatherScatterMode.PROMISE_IN_BOUNDS)


def _make_sc_edge(width, nheads, k, g0_frac=0.5):
    mesh = plsc.VectorSubcoreMesh(core_axis_name="c", subcore_axis_name="s")
    tot = 2 * (EDGES_PER_W // k)
    g0 = max(4, int(round(g0_frac * tot / 4)) * 4)
    g1 = tot - g0
    assert g1 >= 4 and g1 % 4 == 0

    @functools.partial(
        pl.kernel,
        out_type=jax.ShapeDtypeStruct((NC, NPAD, width), jnp.float32),
        mesh=mesh,
        scratch_types=[
            pltpu.VMEM_SHARED((NPAD, width), jnp.float32),
            pltpu.VMEM((4, k), jnp.int32),
            pltpu.VMEM((4, k), jnp.int32),
            pltpu.VMEM((2, k, width), jnp.float32),
            pltpu.VMEM((2, k, 16), jnp.float32),
            pltpu.VMEM((2, k, width), jnp.float32),
            [pltpu.SemaphoreType.DMA] * 2,
            [pltpu.SemaphoreType.DMA] * 2,
            [pltpu.SemaphoreType.DMA] * 2,
            [pltpu.SemaphoreType.DMA] * 4,
        ],
        compiler_params=pltpu.CompilerParams(use_tc_tiling_on_sc=False),
    )
    def sc_edge(src_hbm, dst_hbm, xpa_hbm, adst_hbm, half_hbm, out_hbm,
                acc, sidx, didx, xbuf, abuf, obuf, gx, ga, sc, isem):
        c = lax.axis_index("c")
        s = lax.axis_index("s")
        r0 = s * ROWS_PER_TILE
        row0 = jnp.where(c == 0, s * g0, NS * g0 + s * g1)
        niter = jnp.where(c == 0, g0 // 4, g1 // 4)

        def load_idx(ci, islot):
            pltpu.async_copy(src_hbm.at[row0 + ci], sidx.at[islot],
                             isem[islot])
            pltpu.async_copy(dst_hbm.at[row0 + ci], didx.at[islot],
                             isem[islot])

        def wait_idx(ci, islot):
            pltpu.make_async_copy(src_hbm.at[row0 + ci], sidx.at[islot],
                                  isem[islot]).wait()
            pltpu.make_async_copy(dst_hbm.at[row0 + ci], didx.at[islot],
                                  isem[islot]).wait()

        def issue_gather(islot, bslot):
            pltpu.async_copy(xpa_hbm.at[sidx.at[islot]], xbuf.at[bslot],
                             gx[bslot])
            pltpu.async_copy(adst_hbm.at[didx.at[islot]], abuf.at[bslot],
                             ga[bslot])

        def wait_gather(islot, bslot):
            pltpu.make_async_copy(xpa_hbm.at[sidx.at[islot]], xbuf.at[bslot],
                                  gx[bslot]).wait()
            pltpu.make_async_copy(adst_hbm.at[didx.at[islot]],
                                  abuf.at[bslot], ga[bslot]).wait()

        def issue_scatter(islot, bslot):
            pltpu.async_copy(obuf.at[bslot], acc.at[didx.at[islot]],
                             sc[bslot], add=True)

        def wait_scatter(islot, bslot):
            pltpu.make_async_copy(obuf.at[bslot], acc.at[didx.at[islot]],
                                  sc[bslot]).wait()

        def compute(bslot):
            @plsc.parallel_loop(0, k, 1, unroll=4)
            def edge_body(e):
                av = xbuf[bslot, e, pl.ds(width - 16, 16)] + abuf[bslot, e, :]
                wv = jnp.exp(_lrelu(av))
                obuf[bslot, e, pl.ds(width - 16, 16)] = wv
                for j in range(nheads):
                    ws = _bcast_lane(wv, j)
                    obuf[bslot, e, pl.ds(j * 16, 16)] = (
                        ws * xbuf[bslot, e, pl.ds(j * 16, 16)])

        load_idx(0, 0)
        load_idx(1, 1)
        nstage, rem = divmod(ROWS_PER_TILE, k)
        for t in range(nstage):
            rr = r0 + t * k
            stg = obuf.at[t % 2]
            pltpu.sync_copy(half_hbm.at[pl.ds(rr, k), :], stg)
            pltpu.sync_copy(stg, acc.at[pl.ds(rr, k), :])
        if rem:
            rr = r0 + nstage * k
            stg = obuf.at[nstage % 2]
            pltpu.sync_copy(half_hbm.at[pl.ds(rr, rem), :],
                            stg.at[pl.ds(0, rem), :])
            pltpu.sync_copy(stg.at[pl.ds(0, rem), :],
                            acc.at[pl.ds(rr, rem), :])
        plsc.subcore_barrier()
        wait_idx(0, 0)
        issue_gather(0, 0)

        def quad_body(g, _):
            base = 4 * g
            for j in range(4):
                ci = base + j
                bslot = j % 2
                if j < 3:
                    wait_idx(ci + 1, (j + 1) % 4)
                    issue_gather((j + 1) % 4, 1 - bslot)
                else:
                    @pl.when(g + 1 < niter)
                    def _():
                        wait_idx(ci + 1, (j + 1) % 4)
                        issue_gather((j + 1) % 4, 1 - bslot)

                wait_gather(j, bslot)
                if j < 2:
                    @pl.when(g >= 1)
                    def _():
                        wait_scatter((j + 2) % 4, bslot)
                else:
                    wait_scatter((j + 2) % 4, bslot)
                if j < 2:
                    load_idx(ci + 2, (j + 2) % 4)
                else:
                    @pl.when(g + 1 < niter)
                    def _():
                        load_idx(ci + 2, (j + 2) % 4)

                compute(bslot)
                issue_scatter(j, bslot)
            return 0

        lax.fori_loop(0, niter, quad_body, 0)
        wait_scatter(2, 0)
        wait_scatter(3, 1)
        plsc.subcore_barrier()
        for t in range(nstage):
            rr = r0 + t * k
            stg = obuf.at[t % 2]
            pltpu.sync_copy(acc.at[pl.ds(rr, k), :], stg)
            pltpu.sync_copy(stg, out_hbm.at[c, pl.ds(rr, k), :])
        if rem:
            rr = r0 + nstage * k
            stg = obuf.at[nstage % 2]
            pltpu.sync_copy(acc.at[pl.ds(rr, rem), :],
                            stg.at[pl.ds(0, rem), :])
            pltpu.sync_copy(stg.at[pl.ds(0, rem), :],
                            out_hbm.at[c, pl.ds(rr, rem), :])

    return sc_edge


def _make_sc_edge_deep(width, nheads, k, ring, g0_frac=0.5):
    mesh = plsc.VectorSubcoreMesh(core_axis_name="c", subcore_axis_name="s")
    tot = 2 * (EDGES_PER_W // k)
    g0 = int(round(g0_frac * tot / ring)) * ring
    g1 = tot - g0
    assert g0 >= ring and g1 >= ring and g1 % ring == 0
    chunks = g0

    @functools.partial(
        pl.kernel,
        out_type=jax.ShapeDtypeStruct((NC, NPAD, width), jnp.float32),
        mesh=mesh,
        scratch_types=[
            pltpu.VMEM_SHARED((NPAD, width), jnp.float32),
            pltpu.VMEM((chunks, k), jnp.int32),
            pltpu.VMEM((chunks, k), jnp.int32),
            pltpu.VMEM((ring, k, width), jnp.float32),
            pltpu.VMEM((ring, k, 16), jnp.float32),
            pltpu.VMEM((2, k, width), jnp.float32),
            [pltpu.SemaphoreType.DMA] * ring,
            [pltpu.SemaphoreType.DMA] * ring,
            [pltpu.SemaphoreType.DMA] * 2,
        ],
        compiler_params=pltpu.CompilerParams(use_tc_tiling_on_sc=False),
    )
    def sc_edge(src_hbm, dst_hbm, xpa_hbm, adst_hbm, half_hbm, out_hbm,
                acc, sidx, didx, xbuf, abuf, obuf, gx, ga, sc):
        c = lax.axis_index("c")
        s = lax.axis_index("s")
        r0 = s * ROWS_PER_TILE
        row0 = jnp.where(c == 0, s * g0, NS * g0 + s * g1)
        my_chunks = jnp.where(c == 0, g0, g1)
        niter = my_chunks // ring

        def issue_gather(ci, slot):
            pltpu.async_copy(xpa_hbm.at[sidx.at[ci]], xbuf.at[slot],
                             gx[slot])
            pltpu.async_copy(adst_hbm.at[didx.at[ci]], abuf.at[slot],
                             ga[slot])

        def wait_gather(ci, slot):
            pltpu.make_async_copy(xpa_hbm.at[sidx.at[ci]], xbuf.at[slot],
                                  gx[slot]).wait()
            pltpu.make_async_copy(adst_hbm.at[didx.at[ci]],
                                  abuf.at[slot], ga[slot]).wait()

        def issue_scatter(ci, oslot):
            pltpu.async_copy(obuf.at[oslot], acc.at[didx.at[ci]],
                             sc[oslot], add=True)

        def wait_scatter(ci, oslot):
            pltpu.make_async_copy(obuf.at[oslot], acc.at[didx.at[ci]],
                                  sc[oslot]).wait()

        def compute(slot, oslot):
            @plsc.parallel_loop(0, k, 1, unroll=4)
            def edge_body(e):
                av = xbuf[slot, e, pl.ds(width - 16, 16)] + abuf[slot, e, :]
                wv = jnp.exp(_lrelu(av))
                obuf[oslot, e, pl.ds(width - 16, 16)] = wv
                for j in range(nheads):
                    ws = _bcast_lane(wv, j)
                    obuf[oslot, e, pl.ds(j * 16, 16)] = (
                        ws * xbuf[slot, e, pl.ds(j * 16, 16)])

        @pl.when(c == 0)
        def _():
            pltpu.sync_copy(src_hbm.at[pl.ds(row0, g0), :], sidx)
            pltpu.sync_copy(dst_hbm.at[pl.ds(row0, g0), :], didx)

        @pl.when(c == 1)
        def _():
            pltpu.sync_copy(src_hbm.at[pl.ds(row0, g1), :],
                            sidx.at[pl.ds(0, g1), :])
            pltpu.sync_copy(dst_hbm.at[pl.ds(row0, g1), :],
                            didx.at[pl.ds(0, g1), :])
        nstage, rem = divmod(ROWS_PER_TILE, 2 * k)
        for t in range(nstage):
            rr = r0 + t * 2 * k
            pltpu.sync_copy(half_hbm.at[pl.ds(rr, k), :], obuf.at[0])
            pltpu.sync_copy(half_hbm.at[pl.ds(rr + k, k), :], obuf.at[1])
            pltpu.sync_copy(obuf.at[0], acc.at[pl.ds(rr, k), :])
            pltpu.sync_copy(obuf.at[1], acc.at[pl.ds(rr + k, k), :])
        if rem:
            rr = r0 + nstage * 2 * k
            pltpu.sync_copy(half_hbm.at[pl.ds(rr, rem), :],
                            obuf.at[0].at[pl.ds(0, rem), :])
            pltpu.sync_copy(obuf.at[0].at[pl.ds(0, rem), :],
                            acc.at[pl.ds(rr, rem), :])
        plsc.subcore_barrier()

        for j in range(ring - 2):
            issue_gather(j, j)

        def ring_body(g, _):
            base = ring * g
            for j in range(ring):
                ci = base + j
                oslot = j % 2
                wait_gather(ci, j)
                if j < 2:
                    @pl.when(g >= 1)
                    def _():
                        wait_scatter(ci - 2, oslot)
                else:
                    wait_scatter(ci - 2, oslot)
                nci = ci + ring - 2

                @pl.when(nci < my_chunks)
                def _():
                    issue_gather(nci, (j + ring - 2) % ring)

                compute(j, oslot)
                issue_scatter(ci, oslot)
            return 0

        lax.fori_loop(0, niter, ring_body, 0)
        wait_scatter(0, 0)
        wait_scatter(1, 1)
        plsc.subcore_barrier()
        for t in range(nstage):
            rr = r0 + t * 2 * k
            pltpu.sync_copy(acc.at[pl.ds(rr, k), :], obuf.at[0])
            pltpu.sync_copy(acc.at[pl.ds(rr + k, k), :], obuf.at[1])
            pltpu.sync_copy(obuf.at[0], out_hbm.at[c, pl.ds(rr, k), :])
            pltpu.sync_copy(obuf.at[1], out_hbm.at[c, pl.ds(rr + k, k), :])
        if rem:
            rr = r0 + nstage * 2 * k
            pltpu.sync_copy(acc.at[pl.ds(rr, rem), :],
                            obuf.at[0].at[pl.ds(0, rem), :])
            pltpu.sync_copy(obuf.at[0].at[pl.ds(0, rem), :],
                            out_hbm.at[c, pl.ds(rr, rem), :])

    return sc_edge


_sc_edge1 = _make_sc_edge(ACC1_W, HEADS, 64, g0_frac=0.70)
_sc_edge2 = _make_sc_edge_deep(ACC2_W, 1, 128, 8, g0_frac=0.75)


def _phase_c_body(acc_ref, w2_ref, a2s_ref, a2d_ref, b1_ref,
                  xpa_ref, adst_ref, half_ref):
    num = acc_ref[0, :, 0:W1COLS] + acc_ref[1, :, 0:W1COLS]
    den = acc_ref[0, :, W1COLS:W1COLS + 8] + acc_ref[1, :, W1COLS:W1COLS + 8]
    colh = lax.broadcasted_iota(jnp.int32, (HEADS, W1COLS), 1) // HID
    rowh = lax.broadcasted_iota(jnp.int32, (HEADS, W1COLS), 0)
    rep = (colh == rowh).astype(jnp.float32)
    denr = jnp.dot(den, rep, preferred_element_type=jnp.float32)
    h = num / (denr + 1e-16) + b1_ref[...]
    h = jnp.where(h > 0, h, jnp.exp(jnp.minimum(h, 0.0)) - 1.0)
    xp2 = jnp.dot(h, w2_ref[...], preferred_element_type=jnp.float32)
    asrc2 = jnp.dot(xp2, a2s_ref[...], preferred_element_type=jnp.float32)
    adst2 = jnp.dot(xp2, a2d_ref[...], preferred_element_type=jnp.float32)
    wself = jnp.exp(_lrelu(asrc2 + adst2))
    z15 = jnp.zeros((h.shape[0], 15), jnp.float32)
    xpa_ref[...] = jnp.concatenate([xp2, asrc2, z15], axis=1)
    adst_ref[...] = jnp.concatenate([adst2, z15], axis=1)
    half_ref[...] = jnp.concatenate([xp2 * wself * 0.5, wself * 0.5, z15],
                                    axis=1)


def _phase_c(acc1, W2, a2s, a2d, b1r):
    return pl.pallas_call(
        _phase_c_body,
        grid=(_GRID,),
        in_specs=[
            pl.BlockSpec((NC, _R, ACC1_W), lambda i: (0, i, 0)),
            pl.BlockSpec((W1COLS, NCLS), lambda i: (0, 0)),
            pl.BlockSpec((NCLS, 1), lambda i: (0, 0)),
            pl.BlockSpec((NCLS, 1), lambda i: (0, 0)),
            pl.BlockSpec((1, W1COLS), lambda i: (0, 0)),
        ],
        out_specs=[
            pl.BlockSpec((_R, XPA2_W), lambda i: (i, 0)),
            pl.BlockSpec((_R, 16), lambda i: (i, 0)),
            pl.BlockSpec((_R, XPA2_W), lambda i: (i, 0)),
        ],
        out_shape=[
            jax.ShapeDtypeStruct((N, XPA2_W), jnp.float32),
            jax.ShapeDtypeStruct((N, 16), jnp.float32),
            jax.ShapeDtypeStruct((N, XPA2_W), jnp.float32),
        ],
    )(acc1, W2, a2s, a2d, b1r)


def _phase_e_body(acc_ref, b2_ref, out_ref):
    num = acc_ref[0, :, 0:NCLS] + acc_ref[1, :, 0:NCLS]
    den = acc_ref[0, :, NCLS:NCLS + 1] + acc_ref[1, :, NCLS:NCLS + 1]
    o = num / (den + 1e-16) + b2_ref[...]
    m = jnp.max(o, axis=1, keepdims=True)
    sh = o - m
    lse = jnp.log(jnp.sum(jnp.exp(sh), axis=1, keepdims=True))
    out_ref[...] = sh - lse


def _phase_e(acc2, b2r):
    return pl.pallas_call(
        _phase_e_body,
        grid=(_GRID,),
        in_specs=[
            pl.BlockSpec((NC, _R, ACC2_W), lambda i: (0, i, 0)),
            pl.BlockSpec((1, NCLS), lambda i: (0, 0)),
        ],
        out_specs=pl.BlockSpec((_R, NCLS), lambda i: (i, 0)),
        out_shape=jax.ShapeDtypeStruct((N, NCLS), jnp.float32),
    )(acc2, b2r)


def kernel(x, edge_index, W1, att_src1, att_dst1, b1, W2, att_src2,
           att_dst2, b2):
    src = edge_index[0].astype(jnp.int32)
    dst = edge_index[1].astype(jnp.int32)
    npad_e = EPAD - E
    src_p = jnp.concatenate([src, jnp.zeros((npad_e,), jnp.int32)])
    dst_p = jnp.concatenate([dst, jnp.full((npad_e,), N, jnp.int32)])

    a1s_flat = att_src1.reshape(W1COLS)
    a1d_flat = att_dst1.reshape(W1COLS)
    heads_of = jnp.arange(W1COLS) // HID
    onehot = jax.nn.one_hot(heads_of, HEADS, dtype=jnp.float32)
    A1s = a1s_flat[:, None] * onehot
    A1d = a1d_flat[:, None] * onehot
    a2s = att_src2.reshape(NCLS, 1)
    a2d = att_dst2.reshape(NCLS, 1)
    b1r = b1.reshape(1, W1COLS)
    b2r = b2.reshape(1, NCLS)

    xpa1, adst1, half1 = _phase_a(x, W1, A1s, A1d)
    adst1p = jnp.concatenate(
        [adst1, jnp.zeros((NPAD - N, 16), jnp.float32)], axis=0)
    half1p = jnp.concatenate(
        [half1, jnp.zeros((NPAD - N, ACC1_W), jnp.float32)], axis=0)
    acc1 = _sc_edge1(src_p.reshape(-1, 64), dst_p.reshape(-1, 64),
                     xpa1, adst1p, half1p)

    xpa2, adst2, half2 = _phase_c(acc1, W2, a2s, a2d, b1r)
    adst2p = jnp.concatenate(
        [adst2, jnp.zeros((NPAD - N, 16), jnp.float32)], axis=0)
    half2p = jnp.concatenate(
        [half2, jnp.zeros((NPAD - N, ACC2_W), jnp.float32)], axis=0)
    acc2 = _sc_edge2(src_p.reshape(-1, 128), dst_p.reshape(-1, 128),
                     xpa2, adst2p, half2p)

    return _phase_e(acc2, b2r)
